# baseline (speedup 1.0000x reference)
import dataclasses
import functools

import jax
import jax.numpy as jnp
from jax import lax
from jax.experimental import pallas as pl
from jax.experimental.pallas import tpu as pltpu
from jax.experimental.pallas import tpu_sc as plsc

N = 10000
E = 320000
D = 128
L = 4

NC = 2
NS = 16
NW = NC * NS

IDXW = 128
ROWS_PT = 80
EPT = IDXW * ROWS_PT
E_PAD = NW * EPT

CH = 64
NCH = EPT // CH
NSEC = 4
SEC = NCH // NSEC

N_PAD = 10112
RPT = N_PAD // NS
DROWS = N_PAD // 128

BLK = 128
NBLK = N_PAD // BLK

_mesh = plsc.VectorSubcoreMesh(core_axis_name="c", subcore_axis_name="s")

_sc_cp = pltpu.CompilerParams()
if "needs_layout_passes" in pltpu.CompilerParams.__dataclass_fields__:
    _sc_cp = dataclasses.replace(_sc_cp, needs_layout_passes=False)


@functools.partial(
    pl.kernel,
    out_type=jax.ShapeDtypeStruct((NC, N_PAD, D), jnp.float32),
    mesh=_mesh,
    scratch_types=[
        pltpu.VMEM((SEC, CH), jnp.int32),
        pltpu.VMEM((SEC, CH), jnp.int32),
        pltpu.VMEM((CH, D), jnp.float32),
        pltpu.VMEM((CH, D), jnp.float32),
        pltpu.VMEM((CH, D), jnp.float32),
        pltpu.VMEM((CH, D), jnp.float32),
        pltpu.VMEM_SHARED((N_PAD, D), jnp.float32),
        pltpu.SemaphoreType.DMA,
        pltpu.SemaphoreType.DMA,
        pltpu.SemaphoreType.DMA,
        pltpu.SemaphoreType.DMA,
        pltpu.SemaphoreType.DMA,
        pltpu.SemaphoreType.DMA,
        pltpu.SemaphoreType.DMA,
        pltpu.SemaphoreType.DMA,
    ],
)
def _sc_segsum(h_hbm, src_hbm, dst_hbm, z_hbm, out_hbm,
               sidx, didx, b0, b1, b2, b3, acc,
               g0, g1, g2, g3, s0, s1, s2, s3):
    c = lax.axis_index("c")
    s = lax.axis_index("s")
    base = (c * NS + s) * NCH
    pltpu.sync_copy(z_hbm, acc.at[pl.ds(s * RPT, RPT)])
    plsc.subcore_barrier()

    bufs = (b0, b1, b2, b3)
    gsem = (g0, g1, g2, g3)
    ssem = (s0, s1, s2, s3)

    def gather(t, b):
        pltpu.async_copy(h_hbm.at[sidx.at[t]], bufs[b], gsem[b])

    def gw(t, b):
        pltpu.make_async_copy(h_hbm.at[sidx.at[t]], bufs[b], gsem[b]).wait()

    def scat(t, b):
        pltpu.async_copy(bufs[b], acc.at[didx.at[t]], ssem[b], add=True)

    def scw(t, b):
        pltpu.make_async_copy(bufs[b], acc.at[didx.at[t]], ssem[b]).wait()

    for hf in range(NSEC):
        pltpu.sync_copy(src_hbm.at[pl.ds(base + hf * SEC, SEC)], sidx)
        pltpu.sync_copy(dst_hbm.at[pl.ds(base + hf * SEC, SEC)], didx)

        gather(0, 0)
        gather(1, 1)
        gw(0, 0)
        scat(0, 0)
        gw(1, 1)
        scat(1, 1)
        gather(2, 2)
        gather(3, 3)

        @pl.loop(1, SEC // 2 - 1, step=2)
        def _(kk):
            t = 2 * kk
            gw(t, 2)
            scat(t, 2)
            gw(t + 1, 3)
            scat(t + 1, 3)
            scw(t - 2, 0)
            gather(t + 2, 0)
            scw(t - 1, 1)
            gather(t + 3, 1)
            gw(t + 2, 0)
            scat(t + 2, 0)
            gw(t + 3, 1)
            scat(t + 3, 1)
            scw(t, 2)
            gather(t + 4, 2)
            scw(t + 1, 3)
            gather(t + 5, 3)

        gw(SEC - 2, 2)
        scat(SEC - 2, 2)
        gw(SEC - 1, 3)
        scat(SEC - 1, 3)
        scw(SEC - 4, 0)
        scw(SEC - 3, 1)
        scw(SEC - 2, 2)
        scw(SEC - 1, 3)

    plsc.subcore_barrier()
    pltpu.sync_copy(acc.at[pl.ds(s * RPT, RPT)],
                    out_hbm.at[c, pl.ds(s * RPT, RPT)])


@functools.partial(
    pl.kernel,
    out_type=jax.ShapeDtypeStruct((NW, DROWS, 128), jnp.float32),
    mesh=_mesh,
    compiler_params=_sc_cp,
    scratch_types=[
        pltpu.VMEM((ROWS_PT // 2, IDXW), jnp.int32),
        pltpu.VMEM((DROWS, 128), jnp.float32),
    ],
)
def _sc_degree(dst_hbm, out_hbm, didx, acc):
    c = lax.axis_index("c")
    s = lax.axis_index("s")
    w = c * NS + s
    base = w * ROWS_PT
    HALF = ROWS_PT // 2

    zero16 = jnp.zeros((16,), jnp.float32)
    one16 = jnp.ones((16,), jnp.float32)

    @pl.loop(0, DROWS)
    def _(i):
        for g in range(8):
            acc[i, pl.ds(g * 16, 16)] = zero16

    for hf in range(2):
        pltpu.sync_copy(dst_hbm.at[pl.ds(base + hf * HALF, HALF)], didx)

        @pl.loop(0, HALF)
        def _(r):
            for g in range(8):
                v = didx[r, pl.ds(g * 16, 16)]
                plsc.addupdate_scatter(
                    acc, [lax.shift_right_logical(v, 7),
                          lax.bitwise_and(v, 127)], one16)

    pltpu.sync_copy(acc, out_hbm.at[w])


def _tc_proj(x, W, b):
    def body(x_ref, w_ref, b_ref, o_ref):
        o_ref[...] = (
            jnp.dot(x_ref[...], w_ref[...], preferred_element_type=jnp.float32)
            + b_ref[...]
        )

    return pl.pallas_call(
        body,
        grid=(NBLK,),
        in_specs=[
            pl.BlockSpec((BLK, D), lambda i: (i, 0)),
            pl.BlockSpec((D, D), lambda i: (0, 0)),
            pl.BlockSpec((1, D), lambda i: (0, 0)),
        ],
        out_specs=pl.BlockSpec((BLK, D), lambda i: (i, 0)),
        out_shape=jax.ShapeDtypeStruct((N_PAD, D), jnp.float32),
    )(x, W, b)


def _tc_layer(parts, degs, h, Wl_i, Wr_i, bl_i, g_i, beta_i):
    def body(p_ref, degs_ref, h_ref, wl_ref, wr_ref, bl_ref, g_ref, be_ref,
             o_ref):
        i = pl.program_id(0)
        dgrow = jnp.sum(degs_ref[:, pl.ds(i, 1), :], axis=0)
        inv = 1.0 / jnp.maximum(dgrow, 1.0)
        msg = (p_ref[0] + p_ref[1]) * jnp.transpose(inv)
        out = (
            jnp.dot(msg, wl_ref[...], preferred_element_type=jnp.float32)
            + bl_ref[...]
            + jnp.dot(h_ref[...], wr_ref[...],
                      preferred_element_type=jnp.float32)
        )
        out = out * jax.nn.sigmoid(out)
        mu = jnp.mean(out, axis=1, keepdims=True)
        var = jnp.mean((out - mu) ** 2, axis=1, keepdims=True)
        o_ref[...] = (out - mu) * lax.rsqrt(var + 1e-5) * g_ref[...] \
            + be_ref[...]

    return pl.pallas_call(
        body,
        grid=(NBLK,),
        in_specs=[
            pl.BlockSpec((NC, BLK, D), lambda i: (0, i, 0)),
            pl.BlockSpec((NW, DROWS, 128), lambda i: (0, 0, 0)),
            pl.BlockSpec((BLK, D), lambda i: (i, 0)),
            pl.BlockSpec((D, D), lambda i: (0, 0)),
            pl.BlockSpec((D, D), lambda i: (0, 0)),
            pl.BlockSpec((1, D), lambda i: (0, 0)),
            pl.BlockSpec((1, D), lambda i: (0, 0)),
            pl.BlockSpec((1, D), lambda i: (0, 0)),
        ],
        out_specs=pl.BlockSpec((BLK, D), lambda i: (i, 0)),
        out_shape=jax.ShapeDtypeStruct((N_PAD, D), jnp.float32),
    )(parts, degs, h, Wl_i, Wr_i, bl_i, g_i, beta_i)


def kernel(x, edge_index, W_in, b_in, Wl, bl, Wr, g, beta):
    src = edge_index[0]
    dst = edge_index[1]
    npad = E_PAD - E
    pad_src = jnp.arange(npad, dtype=jnp.int32) % N
    pad_dst = N + jnp.arange(npad, dtype=jnp.int32) % (N_PAD - N)
    src_full = jnp.concatenate([src, pad_src])
    dst_full = jnp.concatenate([dst, pad_dst])
    src2 = src_full.reshape(E_PAD // CH, CH)
    dst2 = dst_full.reshape(E_PAD // CH, CH)
    dst2_deg = dst_full.reshape(E_PAD // IDXW, IDXW)

    zeros_msg = jnp.zeros((RPT, D), jnp.float32)
    x_pad = jnp.pad(x, ((0, N_PAD - N), (0, 0)))

    degs = _sc_degree(dst2_deg)
    h = _tc_proj(x_pad, W_in, b_in.reshape(1, D))
    for i in range(L):
        parts = _sc_segsum(h, src2, dst2, zeros_msg)
        h = _tc_layer(parts, degs, h, Wl[i], Wr[i],
                      bl[i].reshape(1, D), g[i].reshape(1, D),
                      beta[i].reshape(1, D))
    return h[:N]


# speedup vs baseline: 1.3445x; 1.3445x over previous
import dataclasses
import functools

import jax
import jax.numpy as jnp
from jax import lax
from jax.experimental import pallas as pl
from jax.experimental.pallas import tpu as pltpu
from jax.experimental.pallas import tpu_sc as plsc

N = 10000
E = 320000
D = 128
L = 4

NC = 2
NS = 16
NW = NC * NS

IDXW = 128
ROWS_PT = 80
EPT = IDXW * ROWS_PT
E_PAD = NW * EPT

CH = 64
NCH = EPT // CH
NSEC = 4
SEC = NCH // NSEC

N_PAD = 10240
RPT = N_PAD // NS
DROWS = N_PAD // 128

BLK = 1024
NBLK = N_PAD // BLK

_mesh = plsc.VectorSubcoreMesh(core_axis_name="c", subcore_axis_name="s")

_sc_cp = pltpu.CompilerParams()
if "needs_layout_passes" in pltpu.CompilerParams.__dataclass_fields__:
    _sc_cp = dataclasses.replace(_sc_cp, needs_layout_passes=False)


@functools.partial(
    pl.kernel,
    out_type=jax.ShapeDtypeStruct((NC, N_PAD, D), jnp.float32),
    mesh=_mesh,
    scratch_types=[
        pltpu.VMEM((SEC, CH), jnp.int32),
        pltpu.VMEM((SEC, CH), jnp.int32),
        pltpu.VMEM((CH, D), jnp.float32),
        pltpu.VMEM((CH, D), jnp.float32),
        pltpu.VMEM((CH, D), jnp.float32),
        pltpu.VMEM((CH, D), jnp.float32),
        pltpu.VMEM_SHARED((N_PAD, D), jnp.float32),
        pltpu.SemaphoreType.DMA,
        pltpu.SemaphoreType.DMA,
        pltpu.SemaphoreType.DMA,
        pltpu.SemaphoreType.DMA,
        pltpu.SemaphoreType.DMA,
        pltpu.SemaphoreType.DMA,
        pltpu.SemaphoreType.DMA,
        pltpu.SemaphoreType.DMA,
    ],
)
def _sc_segsum(h_hbm, src_hbm, dst_hbm, z_hbm, out_hbm,
               sidx, didx, b0, b1, b2, b3, acc,
               g0, g1, g2, g3, s0, s1, s2, s3):
    c = lax.axis_index("c")
    s = lax.axis_index("s")
    base = (c * NS + s) * NCH
    pltpu.sync_copy(z_hbm, acc.at[pl.ds(s * RPT, RPT)])
    plsc.subcore_barrier()

    bufs = (b0, b1, b2, b3)
    gsem = (g0, g1, g2, g3)
    ssem = (s0, s1, s2, s3)

    def gather(t, b):
        pltpu.async_copy(h_hbm.at[sidx.at[t]], bufs[b], gsem[b])

    def gw(t, b):
        pltpu.make_async_copy(h_hbm.at[sidx.at[t]], bufs[b], gsem[b]).wait()

    def scat(t, b):
        pltpu.async_copy(bufs[b], acc.at[didx.at[t]], ssem[b], add=True)

    def scw(t, b):
        pltpu.make_async_copy(bufs[b], acc.at[didx.at[t]], ssem[b]).wait()

    for hf in range(NSEC):
        pltpu.sync_copy(src_hbm.at[pl.ds(base + hf * SEC, SEC)], sidx)
        pltpu.sync_copy(dst_hbm.at[pl.ds(base + hf * SEC, SEC)], didx)

        gather(0, 0)
        gather(1, 1)
        gw(0, 0)
        scat(0, 0)
        gw(1, 1)
        scat(1, 1)
        gather(2, 2)
        gather(3, 3)

        @pl.loop(1, SEC // 2 - 1, step=2)
        def _(kk):
            t = 2 * kk
            gw(t, 2)
            scat(t, 2)
            gw(t + 1, 3)
            scat(t + 1, 3)
            scw(t - 2, 0)
            gather(t + 2, 0)
            scw(t - 1, 1)
            gather(t + 3, 1)
            gw(t + 2, 0)
            scat(t + 2, 0)
            gw(t + 3, 1)
            scat(t + 3, 1)
            scw(t, 2)
            gather(t + 4, 2)
            scw(t + 1, 3)
            gather(t + 5, 3)

        gw(SEC - 2, 2)
        scat(SEC - 2, 2)
        gw(SEC - 1, 3)
        scat(SEC - 1, 3)
        scw(SEC - 4, 0)
        scw(SEC - 3, 1)
        scw(SEC - 2, 2)
        scw(SEC - 1, 3)

    plsc.subcore_barrier()
    pltpu.sync_copy(acc.at[pl.ds(s * RPT, RPT)],
                    out_hbm.at[c, pl.ds(s * RPT, RPT)])


@functools.partial(
    pl.kernel,
    out_type=jax.ShapeDtypeStruct((NW, DROWS, 128), jnp.float32),
    mesh=_mesh,
    compiler_params=_sc_cp,
    scratch_types=[
        pltpu.VMEM((ROWS_PT // 2, IDXW), jnp.int32),
        pltpu.VMEM((DROWS, 128), jnp.float32),
    ],
)
def _sc_degree(dst_hbm, out_hbm, didx, acc):
    c = lax.axis_index("c")
    s = lax.axis_index("s")
    w = c * NS + s
    base = w * ROWS_PT
    HALF = ROWS_PT // 2

    zero16 = jnp.zeros((16,), jnp.float32)
    one16 = jnp.ones((16,), jnp.float32)

    @pl.loop(0, DROWS)
    def _(i):
        for g in range(8):
            acc[i, pl.ds(g * 16, 16)] = zero16

    for hf in range(2):
        pltpu.sync_copy(dst_hbm.at[pl.ds(base + hf * HALF, HALF)], didx)

        @pl.loop(0, HALF)
        def _(r):
            for g in range(8):
                v = didx[r, pl.ds(g * 16, 16)]
                plsc.addupdate_scatter(
                    acc, [lax.shift_right_logical(v, 7),
                          lax.bitwise_and(v, 127)], one16)

    pltpu.sync_copy(acc, out_hbm.at[w])


def _tc_proj(x, W, b):
    def body(x_ref, w_ref, b_ref, o_ref):
        o_ref[...] = (
            jnp.dot(x_ref[...], w_ref[...], preferred_element_type=jnp.float32)
            + b_ref[...]
        )

    return pl.pallas_call(
        body,
        grid=(NBLK,),
        in_specs=[
            pl.BlockSpec((BLK, D), lambda i: (i, 0)),
            pl.BlockSpec((D, D), lambda i: (0, 0)),
            pl.BlockSpec((1, D), lambda i: (0, 0)),
        ],
        out_specs=pl.BlockSpec((BLK, D), lambda i: (i, 0)),
        out_shape=jax.ShapeDtypeStruct((N_PAD, D), jnp.float32),
    )(x, W, b)


def _tc_layer(parts, degs, h, Wl_i, Wr_i, bl_i, g_i, beta_i):
    def body(p_ref, degs_ref, h_ref, wl_ref, wr_ref, bl_ref, g_ref, be_ref,
             o_ref):
        i = pl.program_id(0)
        dgblk = jnp.sum(degs_ref[:, pl.ds(i * 8, 8), :], axis=0)
        invt = jnp.transpose(1.0 / jnp.maximum(dgblk, 1.0))
        psum = p_ref[0] + p_ref[1]
        msg = jnp.concatenate(
            [psum[k * 128:(k + 1) * 128, :] * invt[:, k:k + 1]
             for k in range(8)], axis=0)
        out = (
            jnp.dot(msg, wl_ref[...], preferred_element_type=jnp.float32)
            + bl_ref[...]
            + jnp.dot(h_ref[...], wr_ref[...],
                      preferred_element_type=jnp.float32)
        )
        out = out * jax.nn.sigmoid(out)
        mu = jnp.mean(out, axis=1, keepdims=True)
        var = jnp.mean((out - mu) ** 2, axis=1, keepdims=True)
        o_ref[...] = (out - mu) * lax.rsqrt(var + 1e-5) * g_ref[...] \
            + be_ref[...]

    return pl.pallas_call(
        body,
        grid=(NBLK,),
        in_specs=[
            pl.BlockSpec((NC, BLK, D), lambda i: (0, i, 0)),
            pl.BlockSpec((NW, DROWS, 128), lambda i: (0, 0, 0)),
            pl.BlockSpec((BLK, D), lambda i: (i, 0)),
            pl.BlockSpec((D, D), lambda i: (0, 0)),
            pl.BlockSpec((D, D), lambda i: (0, 0)),
            pl.BlockSpec((1, D), lambda i: (0, 0)),
            pl.BlockSpec((1, D), lambda i: (0, 0)),
            pl.BlockSpec((1, D), lambda i: (0, 0)),
        ],
        out_specs=pl.BlockSpec((BLK, D), lambda i: (i, 0)),
        out_shape=jax.ShapeDtypeStruct((N_PAD, D), jnp.float32),
    )(parts, degs, h, Wl_i, Wr_i, bl_i, g_i, beta_i)


def kernel(x, edge_index, W_in, b_in, Wl, bl, Wr, g, beta):
    src = edge_index[0]
    dst = edge_index[1]
    npad = E_PAD - E
    pad_src = jnp.arange(npad, dtype=jnp.int32) % N
    pad_dst = N + jnp.arange(npad, dtype=jnp.int32) % (N_PAD - N)
    src_full = jnp.concatenate([src, pad_src])
    dst_full = jnp.concatenate([dst, pad_dst])
    src2 = src_full.reshape(E_PAD // CH, CH)
    dst2 = dst_full.reshape(E_PAD // CH, CH)
    dst2_deg = dst_full.reshape(E_PAD // IDXW, IDXW)

    zeros_msg = jnp.zeros((RPT, D), jnp.float32)
    x_pad = jnp.pad(x, ((0, N_PAD - N), (0, 0)))

    degs = _sc_degree(dst2_deg)
    h = _tc_proj(x_pad, W_in, b_in.reshape(1, D))
    for i in range(L):
        parts = _sc_segsum(h, src2, dst2, zeros_msg)
        h = _tc_layer(parts, degs, h, Wl[i], Wr[i],
                      bl[i].reshape(1, D), g[i].reshape(1, D),
                      beta[i].reshape(1, D))
    return h[:N]
